# initial kernel scaffold (unmeasured)
import jax
import jax.numpy as jnp
from jax import lax
from jax.experimental import pallas as pl
from jax.experimental.pallas import tpu as pltpu

N_DEV = 4
SQ = 1024
SKV = 1024
HQ = 8
DH = 128
D = 1024
SCALE = 0.08838834764831843
NEG = -1e9


def kernel(x, Wq, K_ext, V_ext, Wo):
    x2 = x.reshape(SQ, D)
    k2 = K_ext.reshape(SKV, HQ * DH)
    v2 = V_ext.reshape(SKV, HQ * DH)

    def body(x_ref, wq_ref, k_ref, v_ref, wo_ref, out_ref,
             ctx_buf, lse_buf, ctx_ssem, ctx_rsem, lse_ssem, lse_rsem):
        my = lax.axis_index("i")
        left = (my - 1) % N_DEV
        right = (my + 1) % N_DEV

        barrier = pltpu.get_barrier_semaphore()
        for nbr in (left, right):
            pl.semaphore_signal(barrier, inc=1, device_id=(nbr,),
                                device_id_type=pl.DeviceIdType.MESH)
        pl.semaphore_wait(barrier, 2)

        xb = x_ref[...].astype(jnp.bfloat16)
        wqb = wq_ref[...].astype(jnp.bfloat16)
        q = lax.dot(xb, wqb, preferred_element_type=jnp.float32)

        koff = my * SKV
        qi = lax.broadcasted_iota(jnp.int32, (SQ, SKV), 0)
        kj = lax.broadcasted_iota(jnp.int32, (SQ, SKV), 1) + koff
        mask = (jnp.abs(qi - kj) <= 128) | (kj < 32) | (qi < 32)

        acc = []
        m_run = []
        d_run = []
        for h in range(HQ):
            cols = slice(h * DH, (h + 1) * DH)
            qh = q[:, cols].astype(jnp.bfloat16)
            kh = k_ref[:, cols].astype(jnp.bfloat16)
            s = lax.dot_general(qh, kh, (((1,), (1,)), ((), ())),
                                preferred_element_type=jnp.float32) * SCALE
            s = jnp.where(mask, s, NEG)
            m = jnp.max(s, axis=1, keepdims=True)
            w = jnp.exp(s - m)
            l = jnp.sum(w, axis=1, keepdims=True)
            vh = v_ref[:, cols].astype(jnp.bfloat16)
            n = lax.dot(w.astype(jnp.bfloat16), vh,
                        preferred_element_type=jnp.float32)
            acc.append(n / l)
            m_run.append(m + jnp.log(l))
            d_run.append(jnp.ones_like(m))

        ctx_buf[0] = jnp.concatenate(acc, axis=1).astype(jnp.bfloat16)
        lse_buf[0] = jnp.concatenate(m_run, axis=1)

        for hop in range(N_DEV - 1):
            s_slot = hop % 2
            r_slot = (hop + 1) % 2
            r_ctx = pltpu.make_async_remote_copy(
                src_ref=ctx_buf.at[s_slot], dst_ref=ctx_buf.at[r_slot],
                send_sem=ctx_ssem.at[s_slot], recv_sem=ctx_rsem.at[r_slot],
                device_id=(right,), device_id_type=pl.DeviceIdType.MESH)
            r_lse = pltpu.make_async_remote_copy(
                src_ref=lse_buf.at[s_slot], dst_ref=lse_buf.at[r_slot],
                send_sem=lse_ssem.at[s_slot], recv_sem=lse_rsem.at[r_slot],
                device_id=(right,), device_id_type=pl.DeviceIdType.MESH)
            r_ctx.start()
            r_lse.start()
            r_ctx.wait()
            r_lse.wait()

            ctx_in = ctx_buf[r_slot].astype(jnp.float32)
            lse_in = lse_buf[r_slot]
            for h in range(HQ):
                cols = slice(h * DH, (h + 1) * DH)
                lj = lse_in[:, h:h + 1]
                mn = jnp.maximum(m_run[h], lj)
                a = jnp.exp(m_run[h] - mn)
                b = jnp.exp(lj - mn)
                acc[h] = acc[h] * a + ctx_in[:, cols] * b
                d_run[h] = d_run[h] * a + b
                m_run[h] = mn

        ctx_full = jnp.concatenate(
            [acc[h] / d_run[h] for h in range(HQ)], axis=1
        ).astype(jnp.bfloat16)
        wob = wo_ref[...].astype(jnp.bfloat16)
        out_ref[...] = lax.dot(ctx_full, wob,
                               preferred_element_type=jnp.float32)

    out = pl.pallas_call(
        body,
        out_shape=jax.ShapeDtypeStruct((SQ, D), jnp.float32),
        in_specs=[pl.BlockSpec(memory_space=pltpu.VMEM)] * 5,
        out_specs=pl.BlockSpec(memory_space=pltpu.VMEM),
        scratch_shapes=[
            pltpu.VMEM((2, SQ, HQ * DH), jnp.bfloat16),
            pltpu.VMEM((2, SQ, HQ), jnp.float32),
            pltpu.SemaphoreType.DMA((2,)),
            pltpu.SemaphoreType.DMA((2,)),
            pltpu.SemaphoreType.DMA((2,)),
            pltpu.SemaphoreType.DMA((2,)),
        ],
        compiler_params=pltpu.CompilerParams(collective_id=0),
    )(x2, Wq, k2, v2, Wo)
    return out.reshape(1, SQ, D)


# baseline (device time: 141891 ns/iter reference)
import jax
import jax.numpy as jnp
from jax import lax
from jax.experimental import pallas as pl
from jax.experimental.pallas import tpu as pltpu

N_DEV = 4
SQ = 1024
SKV = 1024
HQ = 8
DH = 128
D = 1024
SCALE = 0.08838834764831843
NEG = -1e9


def kernel(x, Wq, K_ext, V_ext, Wo):
    x2 = x.reshape(SQ, D)
    k2 = K_ext.reshape(SKV, HQ * DH)
    v2 = V_ext.reshape(SKV, HQ * DH)

    def body(x_ref, wq_ref, k_ref, v_ref, wo_ref, out_ref,
             ctx_buf, lse_buf, ctx_ssem, ctx_rsem, lse_ssem, lse_rsem):
        my = lax.axis_index("i")
        left = (my - 1) % N_DEV
        right = (my + 1) % N_DEV

        barrier = pltpu.get_barrier_semaphore()
        for nbr in (left, right):
            pl.semaphore_signal(barrier, inc=1, device_id=(nbr,),
                                device_id_type=pl.DeviceIdType.MESH)
        pl.semaphore_wait(barrier, 2)

        xb = x_ref[...].astype(jnp.bfloat16)
        wqb = wq_ref[...].astype(jnp.bfloat16)
        q = lax.dot(xb, wqb, preferred_element_type=jnp.float32)

        koff = my * SKV
        qi = lax.broadcasted_iota(jnp.int32, (SQ, SKV), 0)
        kj = lax.broadcasted_iota(jnp.int32, (SQ, SKV), 1) + koff
        mask = (jnp.abs(qi - kj) <= 128) | (kj < 32) | (qi < 32)

        acc = []
        m_run = []
        d_run = []
        for h in range(HQ):
            cols = slice(h * DH, (h + 1) * DH)
            qh = q[:, cols].astype(jnp.bfloat16)
            kh = k_ref[:, cols].astype(jnp.bfloat16)
            s = lax.dot_general(qh, kh, (((1,), (1,)), ((), ())),
                                preferred_element_type=jnp.float32) * SCALE
            s = jnp.where(mask, s, NEG)
            m = jnp.max(s, axis=1, keepdims=True)
            w = jnp.exp(s - m)
            l = jnp.sum(w, axis=1, keepdims=True)
            vh = v_ref[:, cols].astype(jnp.bfloat16)
            n = lax.dot(w.astype(jnp.bfloat16), vh,
                        preferred_element_type=jnp.float32)
            acc.append(n / l)
            m_run.append(m + jnp.log(l))
            d_run.append(jnp.ones_like(m))

        ctx_buf[0] = jnp.concatenate(acc, axis=1).astype(jnp.bfloat16)
        lse_buf[0] = jnp.concatenate(m_run, axis=1)

        for hop in range(N_DEV - 1):
            s_slot = hop % 2
            r_slot = (hop + 1) % 2
            r_ctx = pltpu.make_async_remote_copy(
                src_ref=ctx_buf.at[s_slot], dst_ref=ctx_buf.at[r_slot],
                send_sem=ctx_ssem.at[s_slot], recv_sem=ctx_rsem.at[r_slot],
                device_id=(right,), device_id_type=pl.DeviceIdType.MESH)
            r_lse = pltpu.make_async_remote_copy(
                src_ref=lse_buf.at[s_slot], dst_ref=lse_buf.at[r_slot],
                send_sem=lse_ssem.at[s_slot], recv_sem=lse_rsem.at[r_slot],
                device_id=(right,), device_id_type=pl.DeviceIdType.MESH)
            r_ctx.start()
            r_lse.start()
            r_ctx.wait()
            r_lse.wait()

            ctx_in = ctx_buf[r_slot].astype(jnp.float32)
            lse_in = lse_buf[r_slot]
            for h in range(HQ):
                cols = slice(h * DH, (h + 1) * DH)
                lj = lse_in[:, h:h + 1]
                mn = jnp.maximum(m_run[h], lj)
                a = jnp.exp(m_run[h] - mn)
                b = jnp.exp(lj - mn)
                acc[h] = acc[h] * a + ctx_in[:, cols] * b
                d_run[h] = d_run[h] * a + b
                m_run[h] = mn

        ctx_full = jnp.concatenate(
            [acc[h] / d_run[h] for h in range(HQ)], axis=1
        ).astype(jnp.bfloat16)
        wob = wo_ref[...].astype(jnp.bfloat16)
        out_ref[...] = lax.dot(ctx_full, wob,
                               preferred_element_type=jnp.float32)

    out = pl.pallas_call(
        body,
        out_shape=jax.ShapeDtypeStruct((SQ, D), jnp.float32),
        in_specs=[pl.BlockSpec(memory_space=pltpu.VMEM)] * 5,
        out_specs=pl.BlockSpec(memory_space=pltpu.VMEM),
        scratch_shapes=[
            pltpu.VMEM((2, SQ, HQ * DH), jnp.bfloat16),
            pltpu.VMEM((2, SQ, HQ), jnp.float32),
            pltpu.SemaphoreType.DMA((2,)),
            pltpu.SemaphoreType.DMA((2,)),
            pltpu.SemaphoreType.DMA((2,)),
            pltpu.SemaphoreType.DMA((2,)),
        ],
        compiler_params=pltpu.CompilerParams(
            collective_id=0, vmem_limit_bytes=100 * 1024 * 1024),
    )(x2, Wq, k2, v2, Wo)
    return out.reshape(1, SQ, D)


# device time: 52881 ns/iter; 2.6832x vs baseline; 2.6832x over previous
import jax
import jax.numpy as jnp
from jax import lax
from jax.experimental import pallas as pl
from jax.experimental.pallas import tpu as pltpu

N_DEV = 4
SQ = 1024
SKV = 1024
HQ = 8
DH = 128
D = 1024
SCALE = 0.08838834764831843
NEG = -1e9
NB = 160
HALF = 512
G = 1
NG = HQ // G


def _flash(s_pieces, v_pieces):
    m = s_pieces[0].max(axis=1, keepdims=True)
    for s in s_pieces[1:]:
        m = jnp.maximum(m, s.max(axis=1, keepdims=True))
    ws = [jnp.exp(s - m) for s in s_pieces]
    l = sum(w.sum(axis=1, keepdims=True) for w in ws)
    n = sum(
        lax.dot(w.astype(jnp.bfloat16), v, preferred_element_type=jnp.float32)
        for w, v in zip(ws, v_pieces)
    )
    return n / l, m + jnp.log(l)


def kernel(x, Wq, K_ext, V_ext, Wo):

    def body(x_ref, wq_ref, k_ref, v_ref, wo_ref, out_ref,
             xvm, wqvm, kvm, vvm, wovm,
             big_buf, pack_buf, plse_buf,
             in_sems, kv_sems,
             bsend, brecv, fsem, psend, precv, ps2, pr2):
        my = lax.axis_index("i")

        dx = pltpu.make_async_copy(x_ref.at[0], xvm, in_sems.at[0])
        dwq = pltpu.make_async_copy(wq_ref, wqvm, in_sems.at[1])
        dwo = pltpu.make_async_copy(wo_ref, wovm, in_sems.at[2])
        dx.start()
        dwq.start()
        dwo.start()
        kd, vd = [], []
        for h in range(HQ):
            kd.append(pltpu.make_async_copy(
                k_ref.at[0, :, h, :], kvm.at[h], kv_sems.at[0, h]))
            vd.append(pltpu.make_async_copy(
                v_ref.at[0, :, h, :], vvm.at[h], kv_sems.at[1, h]))
            kd[h].start()
            vd[h].start()

        barrier = pltpu.get_barrier_semaphore()
        for nbr in (1, 2, 3):
            pl.semaphore_signal(barrier, inc=1, device_id=((my + nbr) % N_DEV,),
                                device_id_type=pl.DeviceIdType.MESH)
        pl.semaphore_wait(barrier, 3)

        def big_slice(g, half):
            return big_buf.at[pl.ds(G * g, G), pl.ds(32 + half * 432, 432)]

        def big_send(g, half, sem_idx, dev):
            return pltpu.make_async_remote_copy(
                src_ref=big_slice(g, half), dst_ref=big_slice(g, half),
                send_sem=bsend.at[sem_idx, g], recv_sem=brecv.at[half, g],
                device_id=(dev,), device_id_type=pl.DeviceIdType.MESH)

        def big_recv(g, half):
            return pltpu.make_async_remote_copy(
                src_ref=big_slice(g, half), dst_ref=big_slice(g, half),
                send_sem=bsend.at[0, g], recv_sem=brecv.at[half, g],
                device_id=(0,), device_id_type=pl.DeviceIdType.MESH)

        def big_fwd(g, half, dev):
            return pltpu.make_async_remote_copy(
                src_ref=big_slice(g, half), dst_ref=big_slice(g, half),
                send_sem=fsem.at[half, g], recv_sem=brecv.at[half, g],
                device_id=(dev,), device_id_type=pl.DeviceIdType.MESH)

        dx.wait()
        dwq.wait()
        q = lax.dot(xvm[...].astype(jnp.bfloat16),
                    wqvm[...].astype(jnp.bfloat16),
                    preferred_element_type=jnp.float32)

        koff = my * SKV
        qib = lax.broadcasted_iota(jnp.int32, (NB, SKV), 0)
        qib = qib + jnp.where(qib < 32, 0, 864)
        kjb = lax.broadcasted_iota(jnp.int32, (NB, SKV), 1) + koff
        maskb = (jnp.abs(qib - kjb) <= 128) | (kjb < 32) | (qib < 32)

        cnb = []
        lsb = []
        kh_all, vh_all = [], []
        for h in range(HQ):
            cols = slice(h * DH, (h + 1) * DH)
            kd[h].wait()
            vd[h].wait()
            kh_all.append(kvm[h].astype(jnp.bfloat16))
            vh_all.append(vvm[h].astype(jnp.bfloat16))
            qh = jnp.concatenate(
                [q[0:32, cols], q[896:1024, cols]], axis=0).astype(jnp.bfloat16)
            s = lax.dot_general(qh, kh_all[h], (((1,), (1,)), ((), ())),
                                preferred_element_type=jnp.float32) * SCALE
            s = jnp.where(maskb, s, NEG)
            c, ls = _flash([s], [vh_all[h]])
            cnb.append(c)
            lsb.append(ls)

        pk = jnp.concatenate(
            [cnb[h].astype(jnp.bfloat16) for h in range(HQ)], axis=1)
        pls = jnp.concatenate(lsb, axis=1)

        for s_id in range(N_DEV):
            peers = [p for p in range(N_DEV) if p != s_id]

            @pl.when(my == s_id)
            def _():
                pack_buf[s_id] = pk
                plse_buf[s_id] = pls
                for j, p in enumerate(peers):
                    pltpu.make_async_remote_copy(
                        src_ref=pack_buf.at[s_id], dst_ref=pack_buf.at[s_id],
                        send_sem=psend.at[j], recv_sem=precv.at[s_id],
                        device_id=(p,),
                        device_id_type=pl.DeviceIdType.MESH).start()
                    pltpu.make_async_remote_copy(
                        src_ref=plse_buf.at[s_id], dst_ref=plse_buf.at[s_id],
                        send_sem=ps2.at[j], recv_sem=pr2.at[s_id],
                        device_id=(p,),
                        device_id_type=pl.DeviceIdType.MESH).start()

        @pl.when(my == 0)
        def _():
            qi_loc = lax.broadcasted_iota(jnp.int32, (96, 256), 0) + 32
            kj_loc = lax.broadcasted_iota(jnp.int32, (96, 256), 1)
            mask_a = jnp.abs(qi_loc - kj_loc) <= 128
            band_masks = {}
            for b in range(1, 7):
                k0 = (b - 1) * 128
                qi_b = lax.broadcasted_iota(jnp.int32, (128, 384), 0) + b * 128
                kj_b = lax.broadcasted_iota(jnp.int32, (128, 384), 1) + k0
                band_masks[b] = (jnp.abs(qi_b - kj_b) <= 128) | (kj_b < 32)
            strip_mask = (
                lax.broadcasted_iota(jnp.int32, (128, 128), 1) < 32)

            for h in range(HQ):
                cols = slice(h * DH, (h + 1) * DH)
                qh = q[:, cols].astype(jnp.bfloat16)
                kh = kh_all[h]
                vh = vh_all[h]

                s = lax.dot_general(
                    qh[32:128], kh[0:256], (((1,), (1,)), ((), ())),
                    preferred_element_type=jnp.float32) * SCALE
                s = jnp.where(mask_a, s, NEG)
                c, _ = _flash([s], [vh[0:256]])
                big_buf[h, 32:128] = c.astype(jnp.bfloat16)

                for b in range(1, 7):
                    k0 = (b - 1) * 128
                    sb = lax.dot_general(
                        qh[b * 128:(b + 1) * 128], kh[k0:k0 + 384],
                        (((1,), (1,)), ((), ())),
                        preferred_element_type=jnp.float32) * SCALE
                    sb = jnp.where(band_masks[b], sb, NEG)
                    pieces = [sb]
                    vps = [vh[k0:k0 + 384]]
                    if b >= 2:
                        ss = lax.dot_general(
                            qh[b * 128:(b + 1) * 128], kh[0:128],
                            (((1,), (1,)), ((), ())),
                            preferred_element_type=jnp.float32) * SCALE
                        pieces.append(jnp.where(strip_mask, ss, NEG))
                        vps.append(vh[0:128])
                    c, _ = _flash(pieces, vps)
                    big_buf[h, b * 128:(b + 1) * 128] = c.astype(jnp.bfloat16)

                if h % G == G - 1:
                    g = h // G
                    big_send(g, 0, 0, 1).start()
                    big_send(g, 1, 1, 3).start()

        @pl.when(my == 1)
        def _():
            for g in range(NG):
                big_recv(g, 0).wait_recv()
                big_fwd(g, 0, 2).start()

        @pl.when(my == 3)
        def _():
            for g in range(NG):
                big_recv(g, 1).wait_recv()
                big_fwd(g, 1, 2).start()

        @pl.when(my == 2)
        def _():
            for g in range(NG):
                big_recv(g, 0).wait_recv()
                big_fwd(g, 0, 3).start()
                big_recv(g, 1).wait_recv()
                big_fwd(g, 1, 1).start()

        for s_id in range(N_DEV):
            peers = [p for p in range(N_DEV) if p != s_id]

            @pl.when(my == s_id)
            def _():
                for src in peers:
                    pltpu.make_async_remote_copy(
                        src_ref=pack_buf.at[src], dst_ref=pack_buf.at[src],
                        send_sem=psend.at[0], recv_sem=precv.at[src],
                        device_id=(s_id,),
                        device_id_type=pl.DeviceIdType.MESH).wait_recv()
                    pltpu.make_async_remote_copy(
                        src_ref=plse_buf.at[src], dst_ref=plse_buf.at[src],
                        send_sem=ps2.at[0], recv_sem=pr2.at[src],
                        device_id=(s_id,),
                        device_id_type=pl.DeviceIdType.MESH).wait_recv()
                for j in range(3):
                    pltpu.make_async_remote_copy(
                        src_ref=pack_buf.at[s_id], dst_ref=pack_buf.at[s_id],
                        send_sem=psend.at[j], recv_sem=precv.at[s_id],
                        device_id=(s_id,),
                        device_id_type=pl.DeviceIdType.MESH).wait_send()
                    pltpu.make_async_remote_copy(
                        src_ref=plse_buf.at[s_id], dst_ref=plse_buf.at[s_id],
                        send_sem=ps2.at[j], recv_sem=pr2.at[s_id],
                        device_id=(s_id,),
                        device_id_type=pl.DeviceIdType.MESH).wait_send()

        acc = [pack_buf[0][:, h * DH:(h + 1) * DH].astype(jnp.float32)
               for h in range(HQ)]
        m_run = [plse_buf[0][:, h:h + 1] for h in range(HQ)]
        d_run = [jnp.ones_like(m_run[h]) for h in range(HQ)]
        for slot in range(1, N_DEV):
            ctx_in = pack_buf[slot].astype(jnp.float32)
            lse_in = plse_buf[slot]
            for h in range(HQ):
                cols = slice(h * DH, (h + 1) * DH)
                lj = lse_in[:, h:h + 1]
                mn = jnp.maximum(m_run[h], lj)
                a = jnp.exp(m_run[h] - mn)
                b = jnp.exp(lj - mn)
                acc[h] = acc[h] * a + ctx_in[:, cols] * b
                d_run[h] = d_run[h] * a + b
                m_run[h] = mn

        @pl.when(my == 0)
        def _():
            for g in range(NG):
                for i, half in ((0, 0), (1, 1)):
                    pltpu.make_async_remote_copy(
                        src_ref=big_slice(g, half), dst_ref=big_slice(g, half),
                        send_sem=bsend.at[i, g], recv_sem=brecv.at[half, g],
                        device_id=(0,),
                        device_id_type=pl.DeviceIdType.MESH).wait_send()

        @pl.when(my == 1)
        def _():
            for g in range(NG):
                big_recv(g, 1).wait_recv()
                big_fwd(g, 0, 2).wait_send()

        @pl.when(my == 3)
        def _():
            for g in range(NG):
                big_recv(g, 0).wait_recv()
                big_fwd(g, 1, 2).wait_send()

        @pl.when(my == 2)
        def _():
            for g in range(NG):
                big_fwd(g, 0, 3).wait_send()
                big_fwd(g, 1, 1).wait_send()

        comb = jnp.concatenate(
            [acc[h] / d_run[h] for h in range(HQ)], axis=1
        ).astype(jnp.bfloat16)
        big = jnp.concatenate([big_buf[h] for h in range(HQ)], axis=1)
        cfin = jnp.concatenate(
            [comb[0:32], big[32:896], comb[32:NB]], axis=0)
        dwo.wait()
        out_ref[0] = lax.dot(cfin, wovm[...].astype(jnp.bfloat16),
                             preferred_element_type=jnp.float32)

    out = pl.pallas_call(
        body,
        out_shape=jax.ShapeDtypeStruct((1, SQ, D), jnp.float32),
        in_specs=[pl.BlockSpec(memory_space=pl.ANY)] * 5,
        out_specs=pl.BlockSpec(memory_space=pltpu.VMEM),
        scratch_shapes=[
            pltpu.VMEM((SQ, D), jnp.float32),
            pltpu.VMEM((D, D), jnp.float32),
            pltpu.VMEM((HQ, SKV, DH), jnp.float32),
            pltpu.VMEM((HQ, SKV, DH), jnp.float32),
            pltpu.VMEM((D, D), jnp.float32),
            pltpu.VMEM((HQ, SQ, DH), jnp.bfloat16),
            pltpu.VMEM((N_DEV, NB, D), jnp.bfloat16),
            pltpu.VMEM((N_DEV, NB, HQ), jnp.float32),
            pltpu.SemaphoreType.DMA((3,)),
            pltpu.SemaphoreType.DMA((2, HQ)),
            pltpu.SemaphoreType.DMA((4, NG)),
            pltpu.SemaphoreType.DMA((2, NG)),
            pltpu.SemaphoreType.DMA((2, NG)),
            pltpu.SemaphoreType.DMA((3,)),
            pltpu.SemaphoreType.DMA((N_DEV,)),
            pltpu.SemaphoreType.DMA((3,)),
            pltpu.SemaphoreType.DMA((N_DEV,)),
        ],
        compiler_params=pltpu.CompilerParams(
            collective_id=0, vmem_limit_bytes=100 * 1024 * 1024),
    )(x, Wq, K_ext, V_ext, Wo)
    return out
